# baseline (speedup 1.0000x reference)
import functools

import jax
import jax.numpy as jnp
from jax import lax
from jax.experimental import pallas as pl
from jax.experimental.pallas import tpu as pltpu
from jax.experimental.pallas import tpu_sc as plsc

NC = 2
NS = 16
NW = NC * NS
CHUNK = 64
LANES = 16
SKEW = 1.0
NBUF = 4


def _sc_spmm(nodes, d, e_pad, skew):
    total_chunks = e_pad // CHUNK // NS
    q_ = 2 * NBUF
    nc0 = int(round(total_chunks * skew / q_)) * q_
    nc0 = max(q_, min(total_chunks, nc0))
    nc1 = total_chunks - nc0
    assert nc0 % q_ == 0 and nc1 % q_ == 0
    rows_per_tile = nodes // NS
    mesh = plsc.VectorSubcoreMesh(core_axis_name="c", subcore_axis_name="s")

    @functools.partial(
        pl.kernel,
        out_type=jax.ShapeDtypeStruct((NC, nodes, d), jnp.float32),
        mesh=mesh,
        scratch_types=[
            [pltpu.VMEM((CHUNK,), jnp.int32)] * (2 * NBUF),
            [pltpu.VMEM((CHUNK,), jnp.int32)] * (2 * NBUF),
            [pltpu.VMEM((CHUNK,), jnp.float32)] * (2 * NBUF),
            [pltpu.VMEM((CHUNK,), jnp.float32)] * NBUF,
            [pltpu.VMEM((CHUNK, 128), jnp.float32)] * NBUF,
            pltpu.VMEM_SHARED((nodes, 128), jnp.float32),
            [pltpu.SemaphoreType.DMA] * (2 * NBUF),
            [pltpu.SemaphoreType.DMA] * NBUF,
            [pltpu.SemaphoreType.DMA] * NBUF,
        ],
    )
    def spmm(col_hbm, row_hbm, ew_hbm, qn_hbm, hf_hbm, part_hbm,
             col_v, row_v, ew_v, qc_v, rows_v, agg, isem, qsem, hsem):
        c = lax.axis_index("c")
        s = lax.axis_index("s")
        n_chunks = jnp.where(c == 0, nc0, nc1)
        cbase = jnp.where(c == 0, s * nc0, NS * nc0 + s * nc1)

        active = n_chunks > 0

        def zero_body(i, carry):
            for j in range(d // LANES):
                rows_v[0][i, pl.ds(j * LANES, LANES)] = jnp.zeros(
                    (LANES,), jnp.float32)
            return carry
        @pl.when(active)
        def _():
            lax.fori_loop(0, CHUNK, zero_body, 0)
        piece = rows_per_tile
        n_pieces = 1
        while piece > CHUNK:
            n_pieces += 1
            while rows_per_tile % n_pieces:
                n_pieces += 1
            piece = rows_per_tile // n_pieces
        @pl.when(active)
        def _():
            for k in range(n_pieces):
                pltpu.sync_copy(
                    rows_v[0].at[pl.ds(0, piece)],
                    agg.at[pl.ds(s * rows_per_tile + k * piece, piece)])
        plsc.subcore_barrier()

        def issue_idx(kk, b):
            off = (cbase + kk) * CHUNK
            pltpu.async_copy(col_hbm.at[pl.ds(off, CHUNK)], col_v[b], isem[b])
            pltpu.async_copy(row_hbm.at[pl.ds(off, CHUNK)], row_v[b], isem[b])
            pltpu.async_copy(ew_hbm.at[pl.ds(off, CHUNK)], ew_v[b], isem[b])

        def wait_idx(b):
            z = pl.ds(0, CHUNK)
            pltpu.make_async_copy(col_hbm.at[z], col_v[b], isem[b]).wait()
            pltpu.make_async_copy(row_hbm.at[z], row_v[b], isem[b]).wait()
            pltpu.make_async_copy(ew_hbm.at[z], ew_v[b], isem[b]).wait()

        def issue_gathers(db, ib):
            pltpu.async_copy(qn_hbm.at[col_v[ib]], qc_v[db], qsem[db])
            pltpu.async_copy(hf_hbm.at[col_v[ib]], rows_v[db], hsem[db])

        def wait_gathers(db, ib):
            pltpu.make_async_copy(qn_hbm.at[col_v[ib]], qc_v[db], qsem[db]).wait()
            pltpu.make_async_copy(hf_hbm.at[col_v[ib]], rows_v[db], hsem[db]).wait()

        @pl.when(n_chunks > 0)
        def _():
            for b in range(2 * NBUF):
                issue_idx(b, b)
            for b in range(NBUF):
                wait_idx(b)
                issue_gathers(b, b)

        def round_body(k0, carry):
            for b in range(2 * NBUF):
                db = b % NBUF
                kk = k0 * (2 * NBUF) + b
                wait_gathers(db, b)
                def group_body(g, carry2):
                    sl = pl.ds(g * LANES, LANES)
                    coefg = ew_v[b][sl] / qc_v[db][sl]
                    for i in range(LANES):
                        cf = coefg[i]
                        ei = g * LANES + i
                        for j in range(d // LANES):
                            slj = pl.ds(j * LANES, LANES)
                            rows_v[db][ei, slj] = rows_v[db][ei, slj] * cf
                    return carry2
                lax.fori_loop(0, CHUNK // LANES, group_body, 0)
                pltpu.sync_copy(rows_v[db], agg.at[row_v[b]], add=True)
                nb = (b + NBUF) % (2 * NBUF)
                ng = kk + NBUF
                ni = kk + 2 * NBUF

                @pl.when(ng < n_chunks)
                def _():
                    wait_idx(nb)
                    issue_gathers(db, nb)

                @pl.when(ni < n_chunks)
                def _():
                    issue_idx(ni, b)
            return carry
        lax.fori_loop(0, n_chunks // (2 * NBUF), round_body, 0)


        plsc.subcore_barrier()

        @pl.when(active)
        def _():
            pltpu.sync_copy(
                agg.at[pl.ds(s * rows_per_tile, rows_per_tile)],
                part_hbm.at[c, pl.ds(s * rows_per_tile, rows_per_tile)])

    return spmm


def _tc_combine(nodes, d, row_block, ncores):
    def body(p_ref, wt_ref, b_ref, out_ref):
        acc = p_ref[0]
        if ncores > 1:
            acc = acc + p_ref[1]
        out_ref[...] = jnp.dot(
            acc, wt_ref[...], preferred_element_type=jnp.float32) + b_ref[...]

    return pl.pallas_call(
        body,
        grid=(nodes // row_block,),
        in_specs=[
            pl.BlockSpec((NC, row_block, d), lambda i: (0, i, 0)),
            pl.BlockSpec((d, d), lambda i: (0, 0)),
            pl.BlockSpec((1, d), lambda i: (0, 0)),
        ],
        out_specs=pl.BlockSpec((row_block, d), lambda i: (i, 0)),
        out_shape=jax.ShapeDtypeStruct((nodes, d), jnp.float32),
    )


def kernel(edge_index, edge_weight, hidden_feat, num_sampled_nodes, q_probs,
           W, b):
    nodes, d = hidden_feat.shape
    row = edge_index[0].astype(jnp.int32)
    col = edge_index[1].astype(jnp.int32)
    ew = edge_weight.astype(jnp.float32)
    e = row.shape[0]
    step = NW * CHUNK * 2 * NBUF
    e_pad = ((e + step - 1) // step) * step
    pad = e_pad - e
    if pad:
        row = jnp.concatenate([row, jnp.zeros((pad,), jnp.int32)])
        col = jnp.concatenate([col, jnp.zeros((pad,), jnp.int32)])
        ew = jnp.concatenate([ew, jnp.zeros((pad,), jnp.float32)])
    colr, rowr, ewr = col, row, ew
    qn = (q_probs * num_sampled_nodes).astype(jnp.float32)
    nstep = NS * 8
    nodes_pad = ((nodes + nstep - 1) // nstep) * nstep
    npad = nodes_pad - nodes
    hf = hidden_feat.astype(jnp.float32)
    if npad:
        hf = jnp.concatenate([hf, jnp.zeros((npad, d), jnp.float32)])
        qn = jnp.concatenate([qn, jnp.ones((npad,), jnp.float32)])
    part = _sc_spmm(nodes_pad, d, e_pad, SKEW)(colr, rowr, ewr, qn, hf)
    row_block = 1000 if nodes % 1000 == 0 else nodes
    ncores = 2 if SKEW < 1.0 else 1
    out = _tc_combine(nodes, d, row_block, ncores)(
        part, W.T.astype(jnp.float32), b.reshape(1, d).astype(jnp.float32))
    return out


# speedup vs baseline: 2.8745x; 2.8745x over previous
import functools

import jax
import jax.numpy as jnp
from jax import lax
from jax.experimental import pallas as pl
from jax.experimental.pallas import tpu as pltpu
from jax.experimental.pallas import tpu_sc as plsc

NC = 2
NS = 16
NW = NC * NS
CHUNK = 64
LANES = 16
SKEW = 0.5
NBUF = 4


def _sc_spmm(nodes, d, e_pad, skew):
    total_chunks = e_pad // CHUNK // NS
    q_ = 2 * NBUF
    nc0 = int(round(total_chunks * skew / q_)) * q_
    nc0 = max(q_, min(total_chunks, nc0))
    nc1 = total_chunks - nc0
    assert nc0 % q_ == 0 and nc1 % q_ == 0
    rows_per_tile = nodes // NS
    mesh = plsc.VectorSubcoreMesh(core_axis_name="c", subcore_axis_name="s")

    @functools.partial(
        pl.kernel,
        out_type=jax.ShapeDtypeStruct((NC, nodes, d), jnp.float32),
        mesh=mesh,
        scratch_types=[
            [pltpu.VMEM((CHUNK,), jnp.int32)] * (2 * NBUF),
            [pltpu.VMEM((CHUNK,), jnp.int32)] * (2 * NBUF),
            [pltpu.VMEM((CHUNK,), jnp.float32)] * (2 * NBUF),
            [pltpu.VMEM((CHUNK,), jnp.float32)] * NBUF,
            [pltpu.VMEM((CHUNK, 128), jnp.float32)] * NBUF,
            pltpu.VMEM_SHARED((nodes, 128), jnp.float32),
            [pltpu.SemaphoreType.DMA] * (2 * NBUF),
            [pltpu.SemaphoreType.DMA] * NBUF,
            [pltpu.SemaphoreType.DMA] * NBUF,
        ],
    )
    def spmm(col_hbm, row_hbm, ew_hbm, qn_hbm, hf_hbm, part_hbm,
             col_v, row_v, ew_v, qc_v, rows_v, agg, isem, qsem, hsem):
        c = lax.axis_index("c")
        s = lax.axis_index("s")
        n_chunks = jnp.where(c == 0, nc0, nc1)
        cbase = jnp.where(c == 0, s * nc0, NS * nc0 + s * nc1)

        active = n_chunks > 0

        def zero_body(i, carry):
            for j in range(d // LANES):
                rows_v[0][i, pl.ds(j * LANES, LANES)] = jnp.zeros(
                    (LANES,), jnp.float32)
            return carry
        @pl.when(active)
        def _():
            lax.fori_loop(0, CHUNK, zero_body, 0)
        piece = rows_per_tile
        n_pieces = 1
        while piece > CHUNK:
            n_pieces += 1
            while rows_per_tile % n_pieces:
                n_pieces += 1
            piece = rows_per_tile // n_pieces
        @pl.when(active)
        def _():
            for k in range(n_pieces):
                pltpu.sync_copy(
                    rows_v[0].at[pl.ds(0, piece)],
                    agg.at[pl.ds(s * rows_per_tile + k * piece, piece)])
        plsc.subcore_barrier()

        def issue_idx(kk, b):
            off = (cbase + kk) * CHUNK
            pltpu.async_copy(col_hbm.at[pl.ds(off, CHUNK)], col_v[b], isem[b])
            pltpu.async_copy(row_hbm.at[pl.ds(off, CHUNK)], row_v[b], isem[b])
            pltpu.async_copy(ew_hbm.at[pl.ds(off, CHUNK)], ew_v[b], isem[b])

        def wait_idx(b):
            z = pl.ds(0, CHUNK)
            pltpu.make_async_copy(col_hbm.at[z], col_v[b], isem[b]).wait()
            pltpu.make_async_copy(row_hbm.at[z], row_v[b], isem[b]).wait()
            pltpu.make_async_copy(ew_hbm.at[z], ew_v[b], isem[b]).wait()

        def issue_gathers(db, ib):
            pltpu.async_copy(qn_hbm.at[col_v[ib]], qc_v[db], qsem[db])
            pltpu.async_copy(hf_hbm.at[col_v[ib]], rows_v[db], hsem[db])

        def wait_gathers(db, ib):
            pltpu.make_async_copy(qn_hbm.at[col_v[ib]], qc_v[db], qsem[db]).wait()
            pltpu.make_async_copy(hf_hbm.at[col_v[ib]], rows_v[db], hsem[db]).wait()

        @pl.when(n_chunks > 0)
        def _():
            for b in range(2 * NBUF):
                issue_idx(b, b)
            for b in range(NBUF):
                wait_idx(b)
                issue_gathers(b, b)

        def round_body(k0, carry):
            for b in range(2 * NBUF):
                db = b % NBUF
                kk = k0 * (2 * NBUF) + b
                wait_gathers(db, b)
                def group_body(g, carry2):
                    sl = pl.ds(g * LANES, LANES)
                    coefg = ew_v[b][sl] / qc_v[db][sl]
                    for i in range(LANES):
                        cf = coefg[i]
                        ei = g * LANES + i
                        for j in range(d // LANES):
                            slj = pl.ds(j * LANES, LANES)
                            rows_v[db][ei, slj] = rows_v[db][ei, slj] * cf
                    return carry2
                lax.fori_loop(0, CHUNK // LANES, group_body, 0)
                pltpu.sync_copy(rows_v[db], agg.at[row_v[b]], add=True)
                nb = (b + NBUF) % (2 * NBUF)
                ng = kk + NBUF
                ni = kk + 2 * NBUF

                @pl.when(ng < n_chunks)
                def _():
                    wait_idx(nb)
                    issue_gathers(db, nb)

                @pl.when(ni < n_chunks)
                def _():
                    issue_idx(ni, b)
            return carry
        lax.fori_loop(0, n_chunks // (2 * NBUF), round_body, 0)


        plsc.subcore_barrier()

        @pl.when(active)
        def _():
            pltpu.sync_copy(
                agg.at[pl.ds(s * rows_per_tile, rows_per_tile)],
                part_hbm.at[c, pl.ds(s * rows_per_tile, rows_per_tile)])

    return spmm


def _tc_combine(nodes, d, row_block, ncores):
    def body(p_ref, wt_ref, b_ref, out_ref):
        acc = p_ref[0]
        if ncores > 1:
            acc = acc + p_ref[1]
        out_ref[...] = jnp.dot(
            acc, wt_ref[...], preferred_element_type=jnp.float32) + b_ref[...]

    return pl.pallas_call(
        body,
        grid=(nodes // row_block,),
        in_specs=[
            pl.BlockSpec((NC, row_block, d), lambda i: (0, i, 0)),
            pl.BlockSpec((d, d), lambda i: (0, 0)),
            pl.BlockSpec((1, d), lambda i: (0, 0)),
        ],
        out_specs=pl.BlockSpec((row_block, d), lambda i: (i, 0)),
        out_shape=jax.ShapeDtypeStruct((nodes, d), jnp.float32),
    )


def kernel(edge_index, edge_weight, hidden_feat, num_sampled_nodes, q_probs,
           W, b):
    nodes, d = hidden_feat.shape
    row = edge_index[0].astype(jnp.int32)
    col = edge_index[1].astype(jnp.int32)
    ew = edge_weight.astype(jnp.float32)
    e = row.shape[0]
    step = NW * CHUNK * 2 * NBUF
    e_pad = ((e + step - 1) // step) * step
    pad = e_pad - e
    if pad:
        spread = jnp.arange(pad, dtype=jnp.int32) % jnp.int32(nodes)
        row = jnp.concatenate([row, spread])
        col = jnp.concatenate([col, spread])
        ew = jnp.concatenate([ew, jnp.zeros((pad,), jnp.float32)])
    colr, rowr, ewr = col, row, ew
    qn = (q_probs * num_sampled_nodes).astype(jnp.float32)
    nstep = NS * 8
    nodes_pad = ((nodes + nstep - 1) // nstep) * nstep
    npad = nodes_pad - nodes
    hf = hidden_feat.astype(jnp.float32)
    if npad:
        hf = jnp.concatenate([hf, jnp.zeros((npad, d), jnp.float32)])
        qn = jnp.concatenate([qn, jnp.ones((npad,), jnp.float32)])
    part = _sc_spmm(nodes_pad, d, e_pad, SKEW)(colr, rowr, ewr, qn, hf)
    row_block = 1000 if nodes % 1000 == 0 else nodes
    ncores = 2 if SKEW < 1.0 else 1
    out = _tc_combine(nodes, d, row_block, ncores)(
        part, W.T.astype(jnp.float32), b.reshape(1, d).astype(jnp.float32))
    return out
